# baseline (speedup 1.0000x reference)
import jax
import jax.numpy as jnp
from jax.experimental import pallas as pl
from jax.experimental.pallas import tpu as pltpu


def _shift_mask(HW, H, W, dh, dw):
    s = jnp.arange(HW, dtype=jnp.int32)
    h, w = s // W, s % W
    ok = (h + dh >= 0) & (h + dh < H) & (w + dw >= 0) & (w + dw < W)
    return ok.astype(jnp.float32).reshape(1, HW)


def _conv_pass(xin, wmat, bias_m, scale_m, shift_m, alpha_m, masks, *,
               NB, C, HW, W, n_tiles, preop):
    f32 = jnp.float32

    def _body(w_ref, bias_ref, scale_ref, shift_ref, alpha_ref, m_ref,
              x_ref, y_ref, stats_ref):
        acc_s = jnp.zeros((C, HW), f32)
        acc_q = jnp.zeros((C, HW), f32)
        for n in range(NB):
            xi = x_ref[n * C:(n + 1) * C, :].astype(f32)
            if preop:
                z = xi * scale_ref[...] + shift_ref[...]
                a = alpha_ref[0:1, :]
                z = jnp.maximum(z, 0.0) + a * jnp.minimum(z, 0.0)
            else:
                z = xi
            zb = z.astype(jnp.bfloat16)
            pieces = []
            k = 0
            for dh in (-1, 0, 1):
                for dw in (-1, 0, 1):
                    d = W * dh + dw
                    sh = zb if d == 0 else jnp.roll(zb, -d, axis=1)
                    if dh == 0 and dw == 0:
                        pieces.append(sh)
                    else:
                        mk = m_ref[k:k + 1, :].astype(jnp.bfloat16)
                        pieces.append(sh * mk)
                    k += 1
            rhs = jnp.concatenate(pieces, axis=0)
            y = jnp.dot(w_ref[...], rhs, preferred_element_type=f32)
            y = y + bias_ref[...]
            y_ref[n * C:(n + 1) * C, :] = y.astype(jnp.bfloat16)
            acc_s = acc_s + y
            acc_q = acc_q + y * y
        stats_ref[...] = jnp.concatenate([acc_s, acc_q], axis=0)

    return pl.pallas_call(
        _body,
        grid=(n_tiles,),
        in_specs=[
            pl.BlockSpec(wmat.shape, lambda i: (0, 0)),
            pl.BlockSpec((C, HW), lambda i: (0, 0)),
            pl.BlockSpec((C, HW), lambda i: (0, 0)),
            pl.BlockSpec((C, HW), lambda i: (0, 0)),
            pl.BlockSpec((8, HW), lambda i: (0, 0)),
            pl.BlockSpec((16, HW), lambda i: (0, 0)),
            pl.BlockSpec((NB * C, HW), lambda i: (i, 0)),
        ],
        out_specs=[
            pl.BlockSpec((NB * C, HW), lambda i: (i, 0)),
            pl.BlockSpec((2 * C, HW), lambda i: (i, 0)),
        ],
        out_shape=[
            jax.ShapeDtypeStruct(xin.shape, jnp.bfloat16),
            jax.ShapeDtypeStruct((n_tiles * 2 * C, HW), f32),
        ],
        compiler_params=pltpu.CompilerParams(
            dimension_semantics=("parallel",),
            vmem_limit_bytes=100 << 20),
    )(wmat, bias_m, scale_m, shift_m, alpha_m, masks, xin)


def _bn_residual_pass(x, y2, scale_m, shift_m, *, NB, C, HW, W, n_tiles):
    def _body(scale_ref, shift_ref, x_ref, y_ref, o_ref):
        for n in range(NB):
            sl = slice(n * C, (n + 1) * C)
            o_ref[sl, :] = (x_ref[sl, :]
                            + y_ref[sl, :].astype(jnp.float32) * scale_ref[...]
                            + shift_ref[...])

    blk = pl.BlockSpec((NB * C, HW), lambda i: (i, 0))
    return pl.pallas_call(
        _body,
        grid=(n_tiles,),
        in_specs=[pl.BlockSpec((C, HW), lambda i: (0, 0)),
                  pl.BlockSpec((C, HW), lambda i: (0, 0)),
                  blk, blk],
        out_specs=blk,
        out_shape=jax.ShapeDtypeStruct(x.shape, jnp.float32),
        compiler_params=pltpu.CompilerParams(
            dimension_semantics=("parallel",),
            vmem_limit_bytes=100 << 20),
    )(scale_m, shift_m, x, y2)


def _bn_scale_shift(stats, gamma, beta, C, HW, count, eps=1e-5):
    s = jnp.sum(stats.reshape(-1, 2, C, HW), axis=(0, 3))
    mean = s[0] / count
    var = jnp.maximum(s[1] / count - mean * mean, 0.0)
    scale = gamma * jax.lax.rsqrt(var + eps)
    shift = beta - mean * scale
    return scale, shift


def kernel(x, w1, b1, g1, be1, alpha, w2, b2, g2, be2):
    N, C, H, W = x.shape
    HW = H * W
    count = float(N * HW)
    f32 = jnp.float32

    NB = 32
    while N % NB:
        NB //= 2
    n_tiles = N // NB

    xf = x.reshape(N * C, HW)

    wm1 = jnp.transpose(w1, (0, 2, 3, 1)).reshape(C, 9 * C).astype(jnp.bfloat16)
    wm2 = jnp.transpose(w2, (0, 2, 3, 1)).reshape(C, 9 * C).astype(jnp.bfloat16)

    masks = jnp.concatenate(
        [_shift_mask(HW, H, W, dh, dw) for dh in (-1, 0, 1)
         for dw in (-1, 0, 1)] + [jnp.zeros((7, HW), f32)], axis=0)

    def chan_mat(per_channel):
        return jnp.broadcast_to(per_channel.astype(f32)[:, None], (C, HW))

    alpha_m = jnp.broadcast_to(alpha.reshape(()).astype(f32), (8, HW))
    zeros_m = jnp.zeros((C, HW), f32)
    ones_m = jnp.ones((C, HW), f32)

    kw = dict(NB=NB, C=C, HW=HW, W=W, n_tiles=n_tiles)

    y1, st1 = _conv_pass(xf, wm1, chan_mat(b1), ones_m, zeros_m, alpha_m,
                         masks, preop=False, **kw)
    sc1, sh1 = _bn_scale_shift(st1, g1, be1, C, HW, count)

    y2, st2 = _conv_pass(y1, wm2, chan_mat(b2), chan_mat(sc1), chan_mat(sh1),
                         alpha_m, masks, preop=True, **kw)
    sc2, sh2 = _bn_scale_shift(st2, g2, be2, C, HW, count)

    out_flat = _bn_residual_pass(xf, y2, chan_mat(sc2), chan_mat(sh2), **kw)

    return out_flat.reshape(N, C, H, W)


# speedup vs baseline: 1.0104x; 1.0104x over previous
import jax
import jax.numpy as jnp
from jax.experimental import pallas as pl
from jax.experimental.pallas import tpu as pltpu


def _shift_mask(HW, H, W, dh, dw):
    s = jnp.arange(HW, dtype=jnp.int32)
    h, w = s // W, s % W
    ok = (h + dh >= 0) & (h + dh < H) & (w + dw >= 0) & (w + dw < W)
    return ok.astype(jnp.float32).reshape(1, HW)


def _conv_pass(xin, wmat, bias_m, scale_m, shift_m, alpha_m, masks, *,
               NB, C, HW, W, n_tiles, preop):
    f32 = jnp.float32

    def _body(w_ref, bias_ref, scale_ref, shift_ref, alpha_ref, m_ref,
              x_ref, y_ref, stats_ref):
        acc_s = jnp.zeros((C, HW), f32)
        acc_q = jnp.zeros((C, HW), f32)
        for n in range(NB):
            xi = x_ref[n * C:(n + 1) * C, :].astype(f32)
            if preop:
                z = xi * scale_ref[...] + shift_ref[...]
                a = alpha_ref[0:1, :]
                z = jnp.maximum(z, 0.0) + a * jnp.minimum(z, 0.0)
            else:
                z = xi
            zb = z.astype(jnp.bfloat16)
            pieces = []
            k = 0
            for dh in (-1, 0, 1):
                for dw in (-1, 0, 1):
                    d = W * dh + dw
                    sh = zb if d == 0 else jnp.roll(zb, -d, axis=1)
                    if dh == 0 and dw == 0:
                        pieces.append(sh)
                    else:
                        mk = m_ref[k:k + 1, :].astype(jnp.bfloat16)
                        pieces.append(sh * mk)
                    k += 1
            rhs = jnp.concatenate(pieces, axis=0)
            y = jnp.dot(w_ref[...], rhs, preferred_element_type=f32)
            y = y + bias_ref[...]
            y_ref[n * C:(n + 1) * C, :] = y.astype(jnp.bfloat16)
            acc_s = acc_s + y
            acc_q = acc_q + y * y
        stats_ref[...] = jnp.concatenate([acc_s, acc_q], axis=0)

    return pl.pallas_call(
        _body,
        grid=(n_tiles,),
        in_specs=[
            pl.BlockSpec(wmat.shape, lambda i: (0, 0)),
            pl.BlockSpec((C, HW), lambda i: (0, 0)),
            pl.BlockSpec((C, HW), lambda i: (0, 0)),
            pl.BlockSpec((C, HW), lambda i: (0, 0)),
            pl.BlockSpec((8, HW), lambda i: (0, 0)),
            pl.BlockSpec((16, HW), lambda i: (0, 0)),
            pl.BlockSpec((NB * C, HW), lambda i: (i, 0)),
        ],
        out_specs=[
            pl.BlockSpec((NB * C, HW), lambda i: (i, 0)),
            pl.BlockSpec((2 * C, HW), lambda i: (i, 0)),
        ],
        out_shape=[
            jax.ShapeDtypeStruct(xin.shape, jnp.bfloat16),
            jax.ShapeDtypeStruct((n_tiles * 2 * C, HW), f32),
        ],
        compiler_params=pltpu.CompilerParams(
            dimension_semantics=("parallel",),
            vmem_limit_bytes=100 << 20),
    )(wmat, bias_m, scale_m, shift_m, alpha_m, masks, xin)


def _bn_residual_pass(x, y2, scale_m, shift_m, *, NB, C, HW, W, n_tiles):
    NBR = 2 * NB
    n_tiles //= 2

    def _body(scale_ref, shift_ref, x_ref, y_ref, o_ref):
        for n in range(NBR):
            sl = slice(n * C, (n + 1) * C)
            o_ref[sl, :] = (x_ref[sl, :]
                            + y_ref[sl, :].astype(jnp.float32) * scale_ref[...]
                            + shift_ref[...])

    blk = pl.BlockSpec((NBR * C, HW), lambda i: (i, 0))
    return pl.pallas_call(
        _body,
        grid=(n_tiles,),
        in_specs=[pl.BlockSpec((C, HW), lambda i: (0, 0)),
                  pl.BlockSpec((C, HW), lambda i: (0, 0)),
                  blk, blk],
        out_specs=blk,
        out_shape=jax.ShapeDtypeStruct(x.shape, jnp.float32),
        compiler_params=pltpu.CompilerParams(
            dimension_semantics=("parallel",),
            vmem_limit_bytes=100 << 20),
    )(scale_m, shift_m, x, y2)


def _bn_scale_shift(stats, gamma, beta, C, HW, count, eps=1e-5):
    s = jnp.sum(stats.reshape(-1, 2, C, HW), axis=(0, 3))
    mean = s[0] / count
    var = jnp.maximum(s[1] / count - mean * mean, 0.0)
    scale = gamma * jax.lax.rsqrt(var + eps)
    shift = beta - mean * scale
    return scale, shift


def kernel(x, w1, b1, g1, be1, alpha, w2, b2, g2, be2):
    N, C, H, W = x.shape
    HW = H * W
    count = float(N * HW)
    f32 = jnp.float32

    NB = 64
    while N % NB:
        NB //= 2
    n_tiles = N // NB

    xf = x.reshape(N * C, HW)

    wm1 = jnp.transpose(w1, (0, 2, 3, 1)).reshape(C, 9 * C).astype(jnp.bfloat16)
    wm2 = jnp.transpose(w2, (0, 2, 3, 1)).reshape(C, 9 * C).astype(jnp.bfloat16)

    masks = jnp.concatenate(
        [_shift_mask(HW, H, W, dh, dw) for dh in (-1, 0, 1)
         for dw in (-1, 0, 1)] + [jnp.zeros((7, HW), f32)], axis=0)

    def chan_mat(per_channel):
        return jnp.broadcast_to(per_channel.astype(f32)[:, None], (C, HW))

    alpha_m = jnp.broadcast_to(alpha.reshape(()).astype(f32), (8, HW))
    zeros_m = jnp.zeros((C, HW), f32)
    ones_m = jnp.ones((C, HW), f32)

    kw = dict(NB=NB, C=C, HW=HW, W=W, n_tiles=n_tiles)

    y1, st1 = _conv_pass(xf, wm1, chan_mat(b1), ones_m, zeros_m, alpha_m,
                         masks, preop=False, **kw)
    sc1, sh1 = _bn_scale_shift(st1, g1, be1, C, HW, count)

    y2, st2 = _conv_pass(y1, wm2, chan_mat(b2), chan_mat(sc1), chan_mat(sh1),
                         alpha_m, masks, preop=True, **kw)
    sc2, sh2 = _bn_scale_shift(st2, g2, be2, C, HW, count)

    out_flat = _bn_residual_pass(xf, y2, chan_mat(sc2), chan_mat(sh2), **kw)

    return out_flat.reshape(N, C, H, W)


# speedup vs baseline: 1.0256x; 1.0150x over previous
import jax
import jax.numpy as jnp
from jax.experimental import pallas as pl
from jax.experimental.pallas import tpu as pltpu


def _shift_mask(HW, H, W, dh, dw):
    s = jnp.arange(HW, dtype=jnp.int32)
    h, w = s // W, s % W
    ok = (h + dh >= 0) & (h + dh < H) & (w + dw >= 0) & (w + dw < W)
    return ok.astype(jnp.float32).reshape(1, HW)


def _conv_pass(xin, wmat, bias_m, scale_m, shift_m, alpha_m, masks, *,
               NB, C, HW, W, n_tiles, preop):
    f32 = jnp.float32

    def _body(w_ref, bias_ref, scale_ref, shift_ref, alpha_ref, m_ref,
              x_ref, y_ref, stats_ref, *maybe_xb):
        xb_ref = maybe_xb[0] if maybe_xb else None
        acc_s = jnp.zeros((C, HW), f32)
        acc_q = jnp.zeros((C, HW), f32)
        for n in range(NB):
            xi = x_ref[n * C:(n + 1) * C, :].astype(f32)
            if preop:
                z = xi * scale_ref[...] + shift_ref[...]
                a = alpha_ref[0:1, :]
                z = jnp.maximum(z, 0.0) + a * jnp.minimum(z, 0.0)
            else:
                z = xi
            zb = z.astype(jnp.bfloat16)
            if not preop:
                xb_ref[n * C:(n + 1) * C, :] = zb
            pieces = []
            k = 0
            for dh in (-1, 0, 1):
                for dw in (-1, 0, 1):
                    d = W * dh + dw
                    sh = zb if d == 0 else jnp.roll(zb, -d, axis=1)
                    if dh == 0 and dw == 0:
                        pieces.append(sh)
                    else:
                        mk = m_ref[k:k + 1, :].astype(jnp.bfloat16)
                        pieces.append(sh * mk)
                    k += 1
            rhs = jnp.concatenate(pieces, axis=0)
            y = jnp.dot(w_ref[...], rhs, preferred_element_type=f32)
            y = y + bias_ref[...]
            y_ref[n * C:(n + 1) * C, :] = y.astype(jnp.bfloat16)
            acc_s = acc_s + y
            acc_q = acc_q + y * y
        stats_ref[...] = jnp.concatenate([acc_s, acc_q], axis=0)

    return pl.pallas_call(
        _body,
        grid=(n_tiles,),
        in_specs=[
            pl.BlockSpec(wmat.shape, lambda i: (0, 0)),
            pl.BlockSpec((C, HW), lambda i: (0, 0)),
            pl.BlockSpec((C, HW), lambda i: (0, 0)),
            pl.BlockSpec((C, HW), lambda i: (0, 0)),
            pl.BlockSpec((8, HW), lambda i: (0, 0)),
            pl.BlockSpec((16, HW), lambda i: (0, 0)),
            pl.BlockSpec((NB * C, HW), lambda i: (i, 0)),
        ],
        out_specs=[
            pl.BlockSpec((NB * C, HW), lambda i: (i, 0)),
            pl.BlockSpec((2 * C, HW), lambda i: (i, 0)),
        ] + ([] if preop else [pl.BlockSpec((NB * C, HW), lambda i: (i, 0))]),
        out_shape=[
            jax.ShapeDtypeStruct(xin.shape, jnp.bfloat16),
            jax.ShapeDtypeStruct((n_tiles * 2 * C, HW), f32),
        ] + ([] if preop else [jax.ShapeDtypeStruct(xin.shape, jnp.bfloat16)]),
        compiler_params=pltpu.CompilerParams(
            dimension_semantics=("parallel",),
            vmem_limit_bytes=100 << 20),
    )(wmat, bias_m, scale_m, shift_m, alpha_m, masks, xin)


def _bn_residual_pass(x, y2, scale_m, shift_m, *, NB, C, HW, W, n_tiles):
    NBR = 2 * NB
    n_tiles //= 2

    def _body(scale_ref, shift_ref, x_ref, y_ref, o_ref):
        for n in range(NBR):
            sl = slice(n * C, (n + 1) * C)
            o_ref[sl, :] = (x_ref[sl, :].astype(jnp.float32)
                            + y_ref[sl, :].astype(jnp.float32) * scale_ref[...]
                            + shift_ref[...])

    blk = pl.BlockSpec((NBR * C, HW), lambda i: (i, 0))
    return pl.pallas_call(
        _body,
        grid=(n_tiles,),
        in_specs=[pl.BlockSpec((C, HW), lambda i: (0, 0)),
                  pl.BlockSpec((C, HW), lambda i: (0, 0)),
                  blk, blk],
        out_specs=blk,
        out_shape=jax.ShapeDtypeStruct(x.shape, jnp.float32),
        compiler_params=pltpu.CompilerParams(
            dimension_semantics=("parallel",),
            vmem_limit_bytes=100 << 20),
    )(scale_m, shift_m, x, y2)


def _bn_scale_shift(stats, gamma, beta, C, HW, count, eps=1e-5):
    s = jnp.sum(stats.reshape(-1, 2, C, HW), axis=(0, 3))
    mean = s[0] / count
    var = jnp.maximum(s[1] / count - mean * mean, 0.0)
    scale = gamma * jax.lax.rsqrt(var + eps)
    shift = beta - mean * scale
    return scale, shift


def kernel(x, w1, b1, g1, be1, alpha, w2, b2, g2, be2):
    N, C, H, W = x.shape
    HW = H * W
    count = float(N * HW)
    f32 = jnp.float32

    NB = 64
    while N % NB:
        NB //= 2
    n_tiles = N // NB

    xf = x.reshape(N * C, HW)

    wm1 = jnp.transpose(w1, (0, 2, 3, 1)).reshape(C, 9 * C).astype(jnp.bfloat16)
    wm2 = jnp.transpose(w2, (0, 2, 3, 1)).reshape(C, 9 * C).astype(jnp.bfloat16)

    masks = jnp.concatenate(
        [_shift_mask(HW, H, W, dh, dw) for dh in (-1, 0, 1)
         for dw in (-1, 0, 1)] + [jnp.zeros((7, HW), f32)], axis=0)

    def chan_mat(per_channel):
        return jnp.broadcast_to(per_channel.astype(f32)[:, None], (C, HW))

    alpha_m = jnp.broadcast_to(alpha.reshape(()).astype(f32), (8, HW))
    zeros_m = jnp.zeros((C, HW), f32)
    ones_m = jnp.ones((C, HW), f32)

    kw = dict(NB=NB, C=C, HW=HW, W=W, n_tiles=n_tiles)

    y1, st1, xb = _conv_pass(xf, wm1, chan_mat(b1), ones_m, zeros_m, alpha_m,
                             masks, preop=False, **kw)
    sc1, sh1 = _bn_scale_shift(st1, g1, be1, C, HW, count)

    y2, st2 = _conv_pass(y1, wm2, chan_mat(b2), chan_mat(sc1), chan_mat(sh1),
                         alpha_m, masks, preop=True, **kw)
    sc2, sh2 = _bn_scale_shift(st2, g2, be2, C, HW, count)

    out_flat = _bn_residual_pass(xb, y2, chan_mat(sc2), chan_mat(sh2), **kw)

    return out_flat.reshape(N, C, H, W)
